# Initial kernel scaffold; baseline (speedup 1.0000x reference)
import jax
import jax.numpy as jnp
from jax.experimental import pallas as pl


def kernel(x, weight):
    raise NotImplementedError("write your pallas kernel here")


# speedup vs baseline: 10.4022x; 10.4022x over previous
import functools
import itertools

import jax
import jax.numpy as jnp
from jax.experimental import pallas as pl
from jax.experimental.pallas import tpu as pltpu
from jax.experimental.pallas import tpu_sc as plsc

EPS = 1e-8


def _sc_gather(weight, idx_flat, window=256):
    n = idx_flat.shape[0]
    d = weight.shape[1]
    mesh = plsc.VectorSubcoreMesh(core_axis_name="core",
                                  subcore_axis_name="subcore")
    idx2 = idx_flat.reshape(1, n)

    @functools.partial(
        pl.kernel,
        out_type=jax.ShapeDtypeStruct((n, d), weight.dtype),
        mesh=mesh,
        compiler_params=pltpu.CompilerParams(use_tc_tiling_on_sc=False),
    )
    def gather_kernel(w_hbm, i_hbm, o_hbm):
        def body(i_vmem, o_vmem):
            pltpu.sync_copy(w_hbm.at[i_vmem.at[0]], o_vmem)

        pltpu.emit_pipeline(
            body,
            grid=(n // window,),
            in_specs=[pl.BlockSpec((1, window), lambda i: (0, i))],
            out_specs=[pl.BlockSpec((window, d), lambda i: (i, 0))],
            core_axis_name=("core", "subcore"),
            dimension_semantics=(pltpu.PARALLEL,),
        )(i_hbm, o_hbm)

    return gather_kernel(weight, idx2)


def _tc_sims(embed_lbd, pairs, bblk=256):
    l, bn, d = embed_lbd.shape
    p = len(pairs)

    def body(e_ref, o_ref):
        e = [e_ref[i] for i in range(l)]
        na = [jnp.sqrt(jnp.sum(ei * ei, axis=-1, keepdims=True)) for ei in e]
        cols = []
        for i, j in pairs:
            num = jnp.sum(e[i] * e[j], axis=-1, keepdims=True)
            den = jnp.maximum(na[i] * na[j], EPS)
            cols.append(num / den)
        o_ref[...] = jnp.concatenate(cols, axis=-1)

    return pl.pallas_call(
        body,
        grid=(bn // bblk,),
        in_specs=[pl.BlockSpec((l, bblk, d), lambda i: (0, i, 0))],
        out_specs=pl.BlockSpec((bblk, p), lambda i: (i, 0)),
        out_shape=jax.ShapeDtypeStruct((bn, p), jnp.float32),
    )(embed_lbd)


def kernel(x, weight):
    b, l = x.shape
    d = weight.shape[1]
    pairs = list(itertools.combinations(range(l), 2))

    idx_flat = x.T.reshape(-1)
    emb = _sc_gather(weight, idx_flat)
    emb = emb.reshape(l, b, d)
    sims_bp = _tc_sims(emb, pairs)
    return sims_bp.T


# speedup vs baseline: 10.5459x; 1.0138x over previous
import functools
import itertools

import jax
import jax.numpy as jnp
from jax.experimental import pallas as pl
from jax.experimental.pallas import tpu as pltpu
from jax.experimental.pallas import tpu_sc as plsc

EPS = 1e-8


def _sc_gather(weight, idx_flat, window=256):
    n = idx_flat.shape[0]
    d = weight.shape[1]
    mesh = plsc.VectorSubcoreMesh(core_axis_name="core",
                                  subcore_axis_name="subcore")
    idx2 = idx_flat.reshape(1, n)

    @functools.partial(
        pl.kernel,
        out_type=jax.ShapeDtypeStruct((n, d), weight.dtype),
        mesh=mesh,
        compiler_params=pltpu.CompilerParams(use_tc_tiling_on_sc=False),
    )
    def gather_kernel(w_hbm, i_hbm, o_hbm):
        def body(i_vmem, o_vmem):
            pltpu.sync_copy(w_hbm.at[i_vmem.at[0]], o_vmem)

        pltpu.emit_pipeline(
            body,
            grid=(n // window,),
            in_specs=[pl.BlockSpec((1, window), lambda i: (0, i))],
            out_specs=[pl.BlockSpec((window, d), lambda i: (i, 0))],
            core_axis_name=("core", "subcore"),
            dimension_semantics=(pltpu.PARALLEL,),
        )(i_hbm, o_hbm)

    return gather_kernel(weight, idx2)


def _tc_sims(embed_lbd, pairs, bblk=256):
    l, bn, d = embed_lbd.shape
    p = len(pairs)

    def body(e_ref, o_ref):
        e = [e_ref[i] for i in range(l)]
        na = [jnp.sqrt(jnp.sum(ei * ei, axis=-1, keepdims=True)) for ei in e]
        cols = []
        for i, j in pairs:
            num = jnp.sum(e[i] * e[j], axis=-1, keepdims=True)
            den = jnp.maximum(na[i] * na[j], EPS)
            cols.append(num / den)
        o_ref[...] = jnp.concatenate(cols, axis=-1).T

    return pl.pallas_call(
        body,
        grid=(bn // bblk,),
        in_specs=[pl.BlockSpec((l, bblk, d), lambda i: (0, i, 0))],
        out_specs=pl.BlockSpec((p, bblk), lambda i: (0, i)),
        out_shape=jax.ShapeDtypeStruct((p, bn), jnp.float32),
    )(embed_lbd)


def kernel(x, weight):
    b, l = x.shape
    d = weight.shape[1]
    pairs = list(itertools.combinations(range(l), 2))

    idx_flat = x.T.reshape(-1)
    emb = _sc_gather(weight, idx_flat)
    emb = emb.reshape(l, b, d)
    return _tc_sims(emb, pairs)


# speedup vs baseline: 16.5358x; 1.5680x over previous
import functools
import itertools

import jax
import jax.numpy as jnp
from jax.experimental import pallas as pl
from jax.experimental.pallas import tpu as pltpu
from jax.experimental.pallas import tpu_sc as plsc

EPS = 1e-8


def _sc_gather(weight, idx_flat, window=256):
    n = idx_flat.shape[0]
    d = weight.shape[1]
    mesh = plsc.VectorSubcoreMesh(core_axis_name="core",
                                  subcore_axis_name="subcore")
    idx2 = idx_flat.reshape(1, n)

    @functools.partial(
        pl.kernel,
        out_type=jax.ShapeDtypeStruct((n, d), weight.dtype),
        mesh=mesh,
        compiler_params=pltpu.CompilerParams(use_tc_tiling_on_sc=False),
    )
    def gather_kernel(w_hbm, i_hbm, o_hbm):
        def body(i_vmem, o_vmem):
            pltpu.sync_copy(w_hbm.at[i_vmem.at[0]], o_vmem)

        pltpu.emit_pipeline(
            body,
            grid=(n // window,),
            in_specs=[pl.BlockSpec((1, window), lambda i: (0, i))],
            out_specs=[pl.BlockSpec((window, d), lambda i: (i, 0))],
            core_axis_name=("core", "subcore"),
            dimension_semantics=(pltpu.PARALLEL,),
        )(i_hbm, o_hbm)

    return gather_kernel(weight, idx2)


def _tc_sims(embed_lbd, pairs, bblk=256):
    l, bn, d = embed_lbd.shape
    p = len(pairs)

    def body(e_ref, o_ref):
        et = [jnp.transpose(e_ref[i]) for i in range(l)]
        na = [jnp.sqrt(jnp.sum(ei * ei, axis=0, keepdims=True)) for ei in et]
        num = jnp.concatenate(
            [jnp.sum(et[i] * et[j], axis=0, keepdims=True) for i, j in pairs],
            axis=0)
        ni = jnp.concatenate([na[i] for i, _ in pairs], axis=0)
        nj = jnp.concatenate([na[j] for _, j in pairs], axis=0)
        o_ref[...] = num / jnp.maximum(ni * nj, EPS)

    return pl.pallas_call(
        body,
        grid=(bn // bblk,),
        in_specs=[pl.BlockSpec((l, bblk, d), lambda i: (0, i, 0))],
        out_specs=pl.BlockSpec((p, bblk), lambda i: (0, i)),
        out_shape=jax.ShapeDtypeStruct((p, bn), jnp.float32),
    )(embed_lbd)


def kernel(x, weight):
    b, l = x.shape
    d = weight.shape[1]
    pairs = list(itertools.combinations(range(l), 2))

    idx_flat = x.T.reshape(-1)
    emb = _sc_gather(weight, idx_flat)
    emb = emb.reshape(l, b, d)
    return _tc_sims(emb, pairs)


# speedup vs baseline: 18.1643x; 1.0985x over previous
import functools
import itertools

import jax
import jax.numpy as jnp
from jax.experimental import pallas as pl
from jax.experimental.pallas import tpu as pltpu
from jax.experimental.pallas import tpu_sc as plsc

EPS = 1e-8


def _sc_gather(weight, x, window=320):
    b, l = x.shape
    d = weight.shape[1]
    n = b * l
    idx2 = x.reshape(n)
    mesh = plsc.VectorSubcoreMesh(core_axis_name="core",
                                  subcore_axis_name="subcore")

    @functools.partial(
        pl.kernel,
        out_type=jax.ShapeDtypeStruct((n, d), weight.dtype),
        mesh=mesh,
        compiler_params=pltpu.CompilerParams(use_tc_tiling_on_sc=False),
    )
    def gather_kernel(w_hbm, i_hbm, o_hbm):
        def body(i_vmem, o_vmem):
            pltpu.sync_copy(w_hbm.at[i_vmem], o_vmem)

        pltpu.emit_pipeline(
            body,
            grid=(n // window,),
            in_specs=[pl.BlockSpec((window,), lambda i: (i,))],
            out_specs=[pl.BlockSpec((window, d), lambda i: (i, 0))],
            core_axis_name=("core", "subcore"),
            dimension_semantics=(pltpu.PARALLEL,),
        )(i_hbm, o_hbm)

    return gather_kernel(weight, idx2)


def _tc_sims(embed_bld, l, d, pairs, bblk=256):
    bn = embed_bld.shape[0]
    p = len(pairs)

    def body(e_ref, o_ref):
        ebt = jnp.transpose(e_ref[...])
        et = [ebt[i * d:(i + 1) * d] for i in range(l)]
        na = [jnp.sqrt(jnp.sum(ei * ei, axis=0, keepdims=True)) for ei in et]
        num = jnp.concatenate(
            [jnp.sum(et[i] * et[j], axis=0, keepdims=True) for i, j in pairs],
            axis=0)
        ni = jnp.concatenate([na[i] for i, _ in pairs], axis=0)
        nj = jnp.concatenate([na[j] for _, j in pairs], axis=0)
        o_ref[...] = num / jnp.maximum(ni * nj, EPS)

    return pl.pallas_call(
        body,
        grid=(bn // bblk,),
        in_specs=[pl.BlockSpec((bblk, l * d), lambda i: (i, 0))],
        out_specs=pl.BlockSpec((p, bblk), lambda i: (0, i)),
        out_shape=jax.ShapeDtypeStruct((p, bn), jnp.float32),
    )(embed_bld)


def kernel(x, weight):
    b, l = x.shape
    d = weight.shape[1]
    pairs = list(itertools.combinations(range(l), 2))

    emb = _sc_gather(weight, x)
    emb = emb.reshape(b, l * d)
    return _tc_sims(emb, l, d, pairs)


# speedup vs baseline: 25.3204x; 1.3940x over previous
import functools
import itertools

import jax
import jax.numpy as jnp
from jax.experimental import pallas as pl
from jax.experimental.pallas import tpu as pltpu
from jax.experimental.pallas import tpu_sc as plsc

EPS = 1e-8
DPAD = 128


def _tc_prep(weight_t, vblk=8192):
    d, v = weight_t.shape
    grid = (v + vblk - 1) // vblk

    def body(w_ref, o_ref):
        o_ref[:, :d] = jnp.transpose(w_ref[...])

    return pl.pallas_call(
        body,
        grid=(grid,),
        in_specs=[pl.BlockSpec((d, vblk), lambda i: (0, i))],
        out_specs=pl.BlockSpec((vblk, DPAD), lambda i: (i, 0)),
        out_shape=jax.ShapeDtypeStruct((v, DPAD), jnp.float32),
    )(weight_t)


def _sc_gather(table, x_t, window=256):
    l, b = x_t.shape
    n = l * b
    wpr = b // window
    mesh = plsc.VectorSubcoreMesh(core_axis_name="core",
                                  subcore_axis_name="subcore")

    @functools.partial(
        pl.kernel,
        out_type=jax.ShapeDtypeStruct((n, DPAD), jnp.float32),
        mesh=mesh,
    )
    def gather_kernel(w_hbm, i_hbm, o_hbm):
        def body(i_vmem, o_vmem):
            pltpu.sync_copy(w_hbm.at[i_vmem.at[0]], o_vmem)

        pltpu.emit_pipeline(
            body,
            grid=(n // window,),
            in_specs=[pl.BlockSpec((1, window),
                                   lambda i: (i // wpr, i % wpr))],
            out_specs=[pl.BlockSpec((window, DPAD), lambda i: (i, 0))],
            core_axis_name=("core", "subcore"),
            dimension_semantics=(pltpu.PARALLEL,),
        )(i_hbm, o_hbm)

    return gather_kernel(table, x_t)


def _tc_sims(embed_lbd, d, pairs, bblk=512):
    l, bn, dpad = embed_lbd.shape
    p = len(pairs)

    def body(e_ref, o_ref):
        et = [jnp.transpose(e_ref[i])[:d] for i in range(l)]
        na = [jnp.sqrt(jnp.sum(ei * ei, axis=0, keepdims=True)) for ei in et]
        num = jnp.concatenate(
            [jnp.sum(et[i] * et[j], axis=0, keepdims=True) for i, j in pairs],
            axis=0)
        ni = jnp.concatenate([na[i] for i, _ in pairs], axis=0)
        nj = jnp.concatenate([na[j] for _, j in pairs], axis=0)
        o_ref[...] = num / jnp.maximum(ni * nj, EPS)

    return pl.pallas_call(
        body,
        grid=(bn // bblk,),
        in_specs=[pl.BlockSpec((l, bblk, dpad), lambda i: (0, i, 0))],
        out_specs=pl.BlockSpec((p, bblk), lambda i: (0, i)),
        out_shape=jax.ShapeDtypeStruct((p, bn), jnp.float32),
    )(embed_lbd)


def kernel(x, weight):
    b, l = x.shape
    v, d = weight.shape
    pairs = list(itertools.combinations(range(l), 2))

    table = _tc_prep(weight.T)
    emb = _sc_gather(table, x.T)
    emb = emb.reshape(l, b, DPAD)
    return _tc_sims(emb, d, pairs)
